# Initial kernel scaffold; baseline (speedup 1.0000x reference)
import jax
import jax.numpy as jnp
from jax.experimental import pallas as pl


def kernel(w1_flat, b1, w2_flat, b2, x):
    raise NotImplementedError("write your pallas kernel here")


# speedup vs baseline: 2.2020x; 2.2020x over previous
import functools

import jax
import jax.numpy as jnp
from jax import lax
from jax.experimental import pallas as pl
from jax.experimental.pallas import tpu as pltpu


def _block_kernel(B1_ref, B2_ref, b1l_ref, b2l_ref, x_ref, o_ref,
                  xt0_ref, xt1_ref, xt2_ref, yt0_ref, yt1_ref, yt2_ref, *,
                  cin, cmid, cout, depth, height, width):
    H, W = height, width
    xt = [xt0_ref, xt1_ref, xt2_ref]
    yt = [yt0_ref, yt1_ref, yt2_ref]

    def store_shifted(dst_list, dpad, v):
        nlane = v.shape[1]
        zr = jnp.zeros((1, nlane), jnp.float32)
        dst_list[1][dpad] = v
        dst_list[0][dpad, pl.ds(1, W - 1), :] = v[:W - 1]
        dst_list[0][dpad, pl.ds(0, 1), :] = zr
        dst_list[2][dpad, pl.ds(0, W - 1), :] = v[1:]
        dst_list[2][dpad, pl.ds(W - 1, 1), :] = zr

    for dst_list, nch in ((xt, cin), (yt, cmid)):
        z = jnp.zeros((W, nch * H), jnp.float32)
        for r in dst_list:
            r[0] = z
            r[depth + 1] = z

    def fill_x(d, carry):
        v = jnp.concatenate([x_ref[ci, d] for ci in range(cin)], axis=0)
        store_shifted(xt, d + 1, jnp.swapaxes(v, 0, 1))
        return carry

    lax.fori_loop(0, depth, fill_x, 0)

    def conv_mxu(d, src_list, B_ref, bl_ref, n_out):
        acc = None
        for kd in range(3):
            for kw in range(3):
                A = src_list[kw][d + kd]
                t = jnp.dot(A, B_ref[kd * 3 + kw],
                            preferred_element_type=jnp.float32)
                acc = t if acc is None else acc + t
        return jnp.maximum(acc + bl_ref[:, :], 0.0)

    def layer1_step(d, carry):
        y = conv_mxu(d, xt, B1_ref, b1l_ref, cmid)
        store_shifted(yt, d + 1, y)
        return carry

    lax.fori_loop(0, depth, layer1_step, 0)

    def layer2_step(d, carry):
        y = conv_mxu(d, yt, B2_ref, b2l_ref, cout)
        for j in range(cout * H // W):
            t = jnp.swapaxes(y[:, j * W:(j + 1) * W], 0, 1)
            for cc in range(W // H):
                o_ref[j * (W // H) + cc, d] = t[cc * H:(cc + 1) * H, :]
        return carry

    lax.fori_loop(0, depth, layer2_step, 0)


def _band_mats(w_flat, cin_l, cout_l, H):
    w = w_flat.reshape(cout_l, cin_l, 3, 3, 3)
    eyes = jnp.stack([jnp.eye(H, k=1 - kh, dtype=w_flat.dtype)
                      for kh in range(3)])
    mats = []
    for kd in range(3):
        for kw in range(3):
            m = jnp.einsum('oik,kab->iaob', w[:, :, kd, :, kw], eyes)
            mats.append(m.reshape(cin_l * H, cout_l * H))
    return jnp.stack(mats)


def kernel(w1_flat, b1, w2_flat, b2, x):
    N, Cin, D, H, W = x.shape
    Cmid = int(w1_flat.shape[0])
    Cout = int(w2_flat.shape[0])

    B1 = _band_mats(w1_flat, Cin, Cmid, H)
    B2 = _band_mats(w2_flat, Cmid, Cout, H)
    b1l = jnp.repeat(b1, H)[None, :]
    b2l = jnp.repeat(b2, H)[None, :]

    body = functools.partial(
        _block_kernel, cin=Cin, cmid=Cmid, cout=Cout,
        depth=D, height=H, width=W)
    const_spec = lambda shape: pl.BlockSpec(
        shape, lambda n: tuple(0 for _ in shape))

    return pl.pallas_call(
        body,
        out_shape=jax.ShapeDtypeStruct((N, Cout, D, H, W), jnp.float32),
        grid=(N,),
        in_specs=[const_spec((9, Cin * H, Cmid * H)),
                  const_spec((9, Cmid * H, Cout * H)),
                  const_spec((1, Cmid * H)),
                  const_spec((1, Cout * H)),
                  pl.BlockSpec((None, Cin, D, H, W), lambda n: (n, 0, 0, 0, 0))],
        out_specs=pl.BlockSpec((None, Cout, D, H, W), lambda n: (n, 0, 0, 0, 0)),
        scratch_shapes=[pltpu.VMEM((D + 2, W, Cin * H), jnp.float32)] * 3
                      + [pltpu.VMEM((D + 2, W, Cmid * H), jnp.float32)] * 3,
        compiler_params=pltpu.CompilerParams(
            dimension_semantics=("parallel",)),
    )(B1, B2, b1l, b2l, x)


# speedup vs baseline: 2.3697x; 1.0761x over previous
import functools

import jax
import jax.numpy as jnp
from jax import lax
from jax.experimental import pallas as pl
from jax.experimental.pallas import tpu as pltpu


def _block_kernel(B1_ref, B2_ref, b1l_ref, b2l_ref, x_ref, o_ref,
                  xt0_ref, xt1_ref, xt2_ref, yt0_ref, yt1_ref, yt2_ref, *,
                  cin, cmid, cout, depth, height, width):
    H, W = height, width
    xt = [xt0_ref, xt1_ref, xt2_ref]
    yt = [yt0_ref, yt1_ref, yt2_ref]

    def store_shifted(dst_list, dpad, v):
        nlane = v.shape[1]
        zr = jnp.zeros((1, nlane), jnp.float32)
        dst_list[1][dpad] = v
        dst_list[0][dpad, pl.ds(1, W - 1), :] = v[:W - 1]
        dst_list[0][dpad, pl.ds(0, 1), :] = zr
        dst_list[2][dpad, pl.ds(0, W - 1), :] = v[1:]
        dst_list[2][dpad, pl.ds(W - 1, 1), :] = zr

    for dst_list, nch in ((xt, cin), (yt, cmid)):
        z = jnp.zeros((W, nch * H), jnp.float32)
        for r in dst_list:
            r[0] = z
            r[depth + 1] = z

    def fill(d):
        v = jnp.concatenate([x_ref[ci, d] for ci in range(cin)], axis=0)
        store_shifted(xt, d + 1, jnp.swapaxes(v, 0, 1))

    def conv_mxu(d, src_list, B_ref, bl_ref):
        acc = None
        for kd in range(3):
            for kw in range(3):
                A = src_list[kw][d + kd]
                t = jnp.dot(A, B_ref[kd * 3 + kw],
                            preferred_element_type=jnp.float32)
                acc = t if acc is None else acc + t
        return jnp.maximum(acc + bl_ref[:, :], 0.0)

    def l1(d):
        store_shifted(yt, d + 1, conv_mxu(d, xt, B1_ref, b1l_ref))

    def l2(d):
        y = conv_mxu(d, yt, B2_ref, b2l_ref)
        for j in range(cout * H // W):
            t = jnp.swapaxes(y[:, j * W:(j + 1) * W], 0, 1)
            for cc in range(W // H):
                o_ref[j * (W // H) + cc, d] = t[cc * H:(cc + 1) * H, :]

    for i in range(4):
        fill(i)
        if i >= 2:
            l1(i - 2)

    def steady(i, carry):
        fill(i)
        l1(i - 2)
        l2(i - 4)
        return carry

    lax.fori_loop(4, depth, steady, 0)

    for i in range(depth, depth + 4):
        if i - 2 < depth:
            l1(i - 2)
        l2(i - 4)


def _band_mats(w_flat, cin_l, cout_l, H):
    w = w_flat.reshape(cout_l, cin_l, 3, 3, 3)
    eyes = jnp.stack([jnp.eye(H, k=1 - kh, dtype=w_flat.dtype)
                      for kh in range(3)])
    mats = []
    for kd in range(3):
        for kw in range(3):
            m = jnp.einsum('oik,kab->iaob', w[:, :, kd, :, kw], eyes)
            mats.append(m.reshape(cin_l * H, cout_l * H))
    return jnp.stack(mats)


def kernel(w1_flat, b1, w2_flat, b2, x):
    N, Cin, D, H, W = x.shape
    Cmid = int(w1_flat.shape[0])
    Cout = int(w2_flat.shape[0])

    B1 = _band_mats(w1_flat, Cin, Cmid, H)
    B2 = _band_mats(w2_flat, Cmid, Cout, H)
    b1l = jnp.repeat(b1, H)[None, :]
    b2l = jnp.repeat(b2, H)[None, :]

    body = functools.partial(
        _block_kernel, cin=Cin, cmid=Cmid, cout=Cout,
        depth=D, height=H, width=W)
    const_spec = lambda shape: pl.BlockSpec(
        shape, lambda n: tuple(0 for _ in shape))

    return pl.pallas_call(
        body,
        out_shape=jax.ShapeDtypeStruct((N, Cout, D, H, W), jnp.float32),
        grid=(N,),
        in_specs=[const_spec((9, Cin * H, Cmid * H)),
                  const_spec((9, Cmid * H, Cout * H)),
                  const_spec((1, Cmid * H)),
                  const_spec((1, Cout * H)),
                  pl.BlockSpec((None, Cin, D, H, W), lambda n: (n, 0, 0, 0, 0))],
        out_specs=pl.BlockSpec((None, Cout, D, H, W), lambda n: (n, 0, 0, 0, 0)),
        scratch_shapes=[pltpu.VMEM((D + 2, W, Cin * H), jnp.float32)] * 3
                      + [pltpu.VMEM((D + 2, W, Cmid * H), jnp.float32)] * 3,
        compiler_params=pltpu.CompilerParams(
            dimension_semantics=("parallel",)),
    )(B1, B2, b1l, b2l, x)


# speedup vs baseline: 2.6064x; 1.0999x over previous
import functools

import jax
import jax.numpy as jnp
from jax import lax
from jax.experimental import pallas as pl
from jax.experimental.pallas import tpu as pltpu


def _block_kernel(B1_ref, B2_ref, b1l_ref, b2l_ref, x_ref, o_ref,
                  xt_ref, yt_ref, *, cin, cmid, cout, depth, height, width):
    H, W = height, width

    def store_shifted(dst, dpad, v):
        nl = v.shape[1]
        zr = jnp.zeros((1, nl), jnp.bfloat16)
        dst[dpad, :, pl.ds(nl, nl)] = v
        dst[dpad, pl.ds(1, W - 1), pl.ds(0, nl)] = v[:W - 1]
        dst[dpad, pl.ds(0, 1), pl.ds(0, nl)] = zr
        dst[dpad, pl.ds(0, W - 1), pl.ds(2 * nl, nl)] = v[1:]
        dst[dpad, pl.ds(W - 1, 1), pl.ds(2 * nl, nl)] = zr

    for r, nch in ((xt_ref, cin), (yt_ref, cmid)):
        z = jnp.zeros((W, 3 * nch * H), jnp.bfloat16)
        r[0] = z
        r[depth + 1] = z

    def fill(d):
        v = jnp.concatenate([x_ref[ci, d] for ci in range(cin)], axis=0)
        store_shifted(xt_ref, d + 1,
                      jnp.swapaxes(v, 0, 1).astype(jnp.bfloat16))

    def conv_mxu(d, src_ref, B_ref, bl_ref):
        acc = None
        for kd in range(3):
            t = jnp.dot(src_ref[d + kd], B_ref[kd],
                        preferred_element_type=jnp.float32)
            acc = t if acc is None else acc + t
        return jnp.maximum(acc + bl_ref[:, :], 0.0)

    def l1(d):
        y = conv_mxu(d, xt_ref, B1_ref, b1l_ref)
        store_shifted(yt_ref, d + 1, y.astype(jnp.bfloat16))

    def l2(d):
        y = conv_mxu(d, yt_ref, B2_ref, b2l_ref)
        for j in range(cout * H // W):
            t = jnp.swapaxes(y[:, j * W:(j + 1) * W], 0, 1)
            for cc in range(W // H):
                o_ref[j * (W // H) + cc, d] = t[cc * H:(cc + 1) * H, :]

    for i in range(4):
        fill(i)
        if i >= 2:
            l1(i - 2)

    def steady(i, carry):
        fill(i)
        l1(i - 2)
        l2(i - 4)
        return carry

    lax.fori_loop(4, depth, steady, 0)

    for i in range(depth, depth + 4):
        if i - 2 < depth:
            l1(i - 2)
        l2(i - 4)


def _band_mats(w_flat, cin_l, cout_l, H):
    w = w_flat.reshape(cout_l, cin_l, 3, 3, 3)
    eyes = jnp.stack([jnp.eye(H, k=1 - kh, dtype=jnp.float32)
                      for kh in range(3)])
    mats = []
    for kd in range(3):
        m = jnp.einsum('oikw,kab->wiaob', w[:, :, kd], eyes)
        mats.append(m.reshape(3 * cin_l * H, cout_l * H))
    return jnp.stack(mats).astype(jnp.bfloat16)


def kernel(w1_flat, b1, w2_flat, b2, x):
    N, Cin, D, H, W = x.shape
    Cmid = int(w1_flat.shape[0])
    Cout = int(w2_flat.shape[0])

    B1 = _band_mats(w1_flat, Cin, Cmid, H)
    B2 = _band_mats(w2_flat, Cmid, Cout, H)
    b1l = jnp.repeat(b1, H)[None, :]
    b2l = jnp.repeat(b2, H)[None, :]

    body = functools.partial(
        _block_kernel, cin=Cin, cmid=Cmid, cout=Cout,
        depth=D, height=H, width=W)

    const_spec = lambda shape: pl.BlockSpec(
        shape, lambda n: tuple(0 for _ in shape))

    return pl.pallas_call(
        body,
        out_shape=jax.ShapeDtypeStruct((N, Cout, D, H, W), jnp.float32),
        grid=(N,),
        in_specs=[const_spec((3, 3 * Cin * H, Cmid * H)),
                  const_spec((3, 3 * Cmid * H, Cout * H)),
                  const_spec((1, Cmid * H)),
                  const_spec((1, Cout * H)),
                  pl.BlockSpec((None, Cin, D, H, W), lambda n: (n, 0, 0, 0, 0))],
        out_specs=pl.BlockSpec((None, Cout, D, H, W), lambda n: (n, 0, 0, 0, 0)),
        scratch_shapes=[pltpu.VMEM((D + 2, W, 3 * Cin * H), jnp.bfloat16),
                        pltpu.VMEM((D + 2, W, 3 * Cmid * H), jnp.bfloat16)],
        compiler_params=pltpu.CompilerParams(
            dimension_semantics=("parallel",)),
    )(B1, B2, b1l, b2l, x)


# speedup vs baseline: 6.8155x; 2.6149x over previous
import functools

import jax
import jax.numpy as jnp
from jax import lax
from jax.experimental import pallas as pl
from jax.experimental.pallas import tpu as pltpu


def _block_kernel(B1_ref, B2_ref, b1l_ref, b2l_ref, x_ref, o_ref,
                  xt_ref, yt_ref, *, cin, cmid, cout, depth, height, width):
    H, W = height, width

    def scatter_patches(dst, d, v):
        nl = v.shape[1]
        zr = jnp.zeros((1, nl), jnp.bfloat16)
        for kd in range(3):
            r = d + 1 - kd
            if r < 0 or r >= depth:
                continue
            base = 3 * kd * nl
            dst[r, :, pl.ds(base + nl, nl)] = v
            dst[r, pl.ds(1, W - 1), pl.ds(base, nl)] = v[:W - 1]
            dst[r, pl.ds(0, 1), pl.ds(base, nl)] = zr
            dst[r, pl.ds(0, W - 1), pl.ds(base + 2 * nl, nl)] = v[1:]
            dst[r, pl.ds(W - 1, 1), pl.ds(base + 2 * nl, nl)] = zr

    for dst, nch in ((xt_ref, cin), (yt_ref, cmid)):
        z = jnp.zeros((W, 3 * nch * H), jnp.bfloat16)
        dst[0, :, pl.ds(0, 3 * nch * H)] = z
        dst[depth - 1, :, pl.ds(6 * nch * H, 3 * nch * H)] = z

    def fill(d):
        v = jnp.concatenate([x_ref[ci, d] for ci in range(cin)], axis=0)
        scatter_patches(xt_ref, d,
                        jnp.swapaxes(v, 0, 1).astype(jnp.bfloat16))

    def l1(d):
        acc = jnp.dot(xt_ref[d], B1_ref[:, :],
                      preferred_element_type=jnp.float32)
        y = jnp.maximum(acc + b1l_ref[:, :], 0.0)
        scatter_patches(yt_ref, d, y.astype(jnp.bfloat16))

    def l2(d):
        acc = jnp.dot(yt_ref[d], B2_ref[:, :],
                      preferred_element_type=jnp.float32)
        y = jnp.maximum(acc + b2l_ref[:, :], 0.0)
        for j in range(cout * H // W):
            t = jnp.swapaxes(y[:, j * W:(j + 1) * W], 0, 1)
            for cc in range(W // H):
                o_ref[j * (W // H) + cc, d] = t[cc * H:(cc + 1) * H, :]

    G = 4
    fill(0)
    for g0 in range(0, depth + 2 * G, G):
        for i in range(g0, g0 + G):
            if 0 <= i + 1 < depth:
                fill(i + 1)
        for i in range(g0, g0 + G):
            d = i - G + 1
            if 0 <= d < depth:
                l1(d)
        for i in range(g0, g0 + G):
            d = i - 2 * G + 1
            if 0 <= d < depth:
                l2(d)


def _band_mats(w_flat, cin_l, cout_l, H):
    w = w_flat.reshape(cout_l, cin_l, 3, 3, 3)
    eyes = jnp.stack([jnp.eye(H, k=1 - kh, dtype=jnp.float32)
                      for kh in range(3)])
    m = jnp.einsum('oidkw,kab->dwiaob', w, eyes)
    return m.reshape(9 * cin_l * H, cout_l * H).astype(jnp.bfloat16)


def kernel(w1_flat, b1, w2_flat, b2, x):
    N, Cin, D, H, W = x.shape
    Cmid = int(w1_flat.shape[0])
    Cout = int(w2_flat.shape[0])

    B1 = _band_mats(w1_flat, Cin, Cmid, H)
    B2 = _band_mats(w2_flat, Cmid, Cout, H)
    b1l = jnp.repeat(b1, H)[None, :]
    b2l = jnp.repeat(b2, H)[None, :]

    body = functools.partial(
        _block_kernel, cin=Cin, cmid=Cmid, cout=Cout,
        depth=D, height=H, width=W)

    const_spec = lambda shape: pl.BlockSpec(
        shape, lambda n: tuple(0 for _ in shape))

    return pl.pallas_call(
        body,
        out_shape=jax.ShapeDtypeStruct((N, Cout, D, H, W), jnp.float32),
        grid=(N,),
        in_specs=[const_spec((9 * Cin * H, Cmid * H)),
                  const_spec((9 * Cmid * H, Cout * H)),
                  const_spec((1, Cmid * H)),
                  const_spec((1, Cout * H)),
                  pl.BlockSpec((None, Cin, D, H, W), lambda n: (n, 0, 0, 0, 0))],
        out_specs=pl.BlockSpec((None, Cout, D, H, W), lambda n: (n, 0, 0, 0, 0)),
        scratch_shapes=[pltpu.VMEM((D, W, 9 * Cin * H), jnp.bfloat16),
                        pltpu.VMEM((D, W, 9 * Cmid * H), jnp.bfloat16)],
        compiler_params=pltpu.CompilerParams(
            dimension_semantics=("parallel",)),
    )(B1, B2, b1l, b2l, x)


# speedup vs baseline: 6.8166x; 1.0002x over previous
import functools

import jax
import jax.numpy as jnp
from jax import lax
from jax.experimental import pallas as pl
from jax.experimental.pallas import tpu as pltpu


def _block_kernel(B1_ref, B2_ref, b1l_ref, b2l_ref, x_ref, o_ref,
                  xt_ref, yt_ref, *, cin, cmid, cout, depth, height, width):
    H, W = height, width

    def scatter_patches(dst, d, v):
        nl = v.shape[1]
        zr = jnp.zeros((1, nl), jnp.bfloat16)
        for kd in range(3):
            r = d + 1 - kd
            if r < 0 or r >= depth:
                continue
            base = 3 * kd * nl
            dst[r, :, pl.ds(base + nl, nl)] = v
            dst[r, pl.ds(1, W - 1), pl.ds(base, nl)] = v[:W - 1]
            dst[r, pl.ds(0, 1), pl.ds(base, nl)] = zr
            dst[r, pl.ds(0, W - 1), pl.ds(base + 2 * nl, nl)] = v[1:]
            dst[r, pl.ds(W - 1, 1), pl.ds(base + 2 * nl, nl)] = zr

    for dst, nch in ((xt_ref, cin), (yt_ref, cmid)):
        z = jnp.zeros((W, 3 * nch * H), jnp.bfloat16)
        dst[0, :, pl.ds(0, 3 * nch * H)] = z
        dst[depth - 1, :, pl.ds(6 * nch * H, 3 * nch * H)] = z

    def fill(d):
        v = jnp.concatenate([x_ref[ci, d] for ci in range(cin)], axis=0)
        scatter_patches(xt_ref, d,
                        jnp.swapaxes(v, 0, 1).astype(jnp.bfloat16))

    def l1(d):
        acc = jnp.dot(xt_ref[d], B1_ref[:, :],
                      preferred_element_type=jnp.float32)
        y = jnp.maximum(acc + b1l_ref[:, :], 0.0)
        scatter_patches(yt_ref, d, y.astype(jnp.bfloat16))

    def l2(d):
        acc = jnp.dot(yt_ref[d], B2_ref[:, :],
                      preferred_element_type=jnp.float32)
        y = jnp.maximum(acc + b2l_ref[:, :], 0.0)
        for j in range(cout * H // W):
            t = jnp.swapaxes(y[:, j * W:(j + 1) * W], 0, 1)
            for cc in range(W // H):
                o_ref[j * (W // H) + cc, d] = t[cc * H:(cc + 1) * H, :]

    G = 8
    fill(0)
    for g0 in range(0, depth + 2 * G, G):
        for i in range(g0, g0 + G):
            if 0 <= i + 1 < depth:
                fill(i + 1)
        for i in range(g0, g0 + G):
            d = i - G + 1
            if 0 <= d < depth:
                l1(d)
        for i in range(g0, g0 + G):
            d = i - 2 * G + 1
            if 0 <= d < depth:
                l2(d)


def _band_mats(w_flat, cin_l, cout_l, H):
    w = w_flat.reshape(cout_l, cin_l, 3, 3, 3)
    eyes = jnp.stack([jnp.eye(H, k=1 - kh, dtype=jnp.float32)
                      for kh in range(3)])
    m = jnp.einsum('oidkw,kab->dwiaob', w, eyes)
    return m.reshape(9 * cin_l * H, cout_l * H).astype(jnp.bfloat16)


def kernel(w1_flat, b1, w2_flat, b2, x):
    N, Cin, D, H, W = x.shape
    Cmid = int(w1_flat.shape[0])
    Cout = int(w2_flat.shape[0])

    B1 = _band_mats(w1_flat, Cin, Cmid, H)
    B2 = _band_mats(w2_flat, Cmid, Cout, H)
    b1l = jnp.repeat(b1, H)[None, :]
    b2l = jnp.repeat(b2, H)[None, :]

    body = functools.partial(
        _block_kernel, cin=Cin, cmid=Cmid, cout=Cout,
        depth=D, height=H, width=W)

    const_spec = lambda shape: pl.BlockSpec(
        shape, lambda n: tuple(0 for _ in shape))

    return pl.pallas_call(
        body,
        out_shape=jax.ShapeDtypeStruct((N, Cout, D, H, W), jnp.float32),
        grid=(N,),
        in_specs=[const_spec((9 * Cin * H, Cmid * H)),
                  const_spec((9 * Cmid * H, Cout * H)),
                  const_spec((1, Cmid * H)),
                  const_spec((1, Cout * H)),
                  pl.BlockSpec((None, Cin, D, H, W), lambda n: (n, 0, 0, 0, 0))],
        out_specs=pl.BlockSpec((None, Cout, D, H, W), lambda n: (n, 0, 0, 0, 0)),
        scratch_shapes=[pltpu.VMEM((D, W, 9 * Cin * H), jnp.bfloat16),
                        pltpu.VMEM((D, W, 9 * Cmid * H), jnp.bfloat16)],
        compiler_params=pltpu.CompilerParams(
            dimension_semantics=("parallel",)),
    )(B1, B2, b1l, b2l, x)
